# baseline (speedup 1.0000x reference)
import functools

import jax
import jax.numpy as jnp
from jax import lax
from jax.experimental import pallas as pl
from jax.experimental.pallas import tpu as pltpu
from jax.experimental.pallas import tpu_sc as plsc

NC = 2
NS = 16
NW = NC * NS
CHUNK = 128
F32 = jnp.float32


def _sc_mesh():
    return plsc.VectorSubcoreMesh(
        core_axis_name="c", subcore_axis_name="s", num_cores=NC, num_subcores=NS
    )


def _deg_kernel(NPAD, TPB):
    SEG = NPAD // NS

    @functools.partial(
        pl.kernel,
        out_type=jax.ShapeDtypeStruct((NC, NPAD), F32),
        mesh=_sc_mesh(),
        scratch_types=[
            pltpu.VMEM((TPB, CHUNK), jnp.int32),
            pltpu.VMEM((CHUNK,), F32),
            pltpu.VMEM((SEG,), F32),
            pltpu.VMEM_SHARED((NPAD,), F32),
        ],
    )
    def deg(dst_hbm, out_hbm, dst_v, ones_v, seg_v, deg_sh):
        cid = lax.axis_index("c")
        sid = lax.axis_index("s")
        wid = cid * NS + sid
        pltpu.sync_copy(dst_hbm.at[wid], dst_v)

        zero16 = jnp.zeros((16,), F32)
        one16 = jnp.ones((16,), F32)
        for k in range(CHUNK // 16):
            ones_v[pl.ds(k * 16, 16)] = one16

        def zbody(q, carry):
            seg_v[pl.ds(q * 16, 16)] = zero16
            return carry

        lax.fori_loop(0, SEG // 16, zbody, 0)
        seg0 = sid * SEG
        sl = pl.ds(seg0, SEG)
        pltpu.sync_copy(seg_v, deg_sh.at[sl])
        plsc.subcore_barrier()

        def cbody(j, carry):
            pltpu.sync_copy(ones_v, deg_sh.at[dst_v.at[j]], add=True)
            return carry

        lax.fori_loop(0, TPB, cbody, 0)
        plsc.subcore_barrier()

        pltpu.sync_copy(deg_sh.at[sl], seg_v)
        pltpu.sync_copy(seg_v, out_hbm.at[cid, sl])

    return deg


def _msg_kernel(N, D, NPAD, TPB2):
    DH = D // NC
    RPT = NPAD // NS
    WCHUNKS = RPT // CHUNK

    NV = TPB2
    NB = 4
    LA = 3
    BF16 = jnp.bfloat16

    @functools.partial(
        pl.kernel,
        out_type=jax.ShapeDtypeStruct((NC, NPAD, DH), F32),
        mesh=_sc_mesh(),
        compiler_params=pltpu.CompilerParams(
            use_tc_tiling_on_sc=False, needs_layout_passes=False
        ),
        scratch_types=[
            pltpu.VMEM((NV, CHUNK), jnp.int32),
            pltpu.VMEM((NV, CHUNK), jnp.int32),
            pltpu.VMEM((NB, CHUNK, DH), BF16),
            pltpu.VMEM((NB, CHUNK, DH), F32),
            pltpu.VMEM_SHARED((NPAD, DH), F32),
            pltpu.SemaphoreType.DMA((NB,)),
            pltpu.SemaphoreType.DMA((NB,)),
        ],
    )
    def msg(g_hbm, src_hbm, dst_hbm, zeros_hbm, out_hbm,
            src_v, dst_v, bbuf, fbuf, acc_sh, gsem, ssem):
        cid = lax.axis_index("c")
        sid = lax.axis_index("s")
        gt = g_hbm

        pltpu.sync_copy(src_hbm.at[sid], src_v)
        pltpu.sync_copy(dst_hbm.at[sid], dst_v)

        cvec = jnp.full((16,), NC, jnp.int32)
        ovec = jnp.zeros((16,), jnp.int32) + cid

        def xbody(j, carry):
            for k in range(CHUNK // 16):
                sl16 = pl.ds(k * 16, 16)
                src_v[j, sl16] = src_v[j, sl16] * cvec + ovec
            return carry

        lax.fori_loop(0, NV, xbody, 0)

        base = sid * RPT
        pltpu.sync_copy(zeros_hbm, fbuf.at[0])
        for r in range(WCHUNKS):
            pltpu.sync_copy(fbuf.at[0], acc_sh.at[pl.ds(base + r * CHUNK, CHUNK)])
        plsc.subcore_barrier()

        def gather(j, b):
            pltpu.async_copy(gt.at[src_v.at[j]], bbuf.at[b], gsem.at[b])

        def wait_gather(j, b):
            pltpu.make_async_copy(
                gt.at[src_v.at[j]], bbuf.at[b], gsem.at[b]
            ).wait()

        def scatter(j, b):
            pltpu.async_copy(
                fbuf.at[b], acc_sh.at[dst_v.at[j]], ssem.at[b], add=True
            )

        def wait_scatter(j, b):
            pltpu.make_async_copy(
                fbuf.at[b], acc_sh.at[dst_v.at[j]], ssem.at[b]
            ).wait()

        def convert(b):
            def cbody(r, carry):
                for k in range(DH // 32):
                    v = bbuf[b, r, pl.ds(k * 32, 32)]
                    lo, hi = plsc.unpack(v, format=plsc.PackFormat.INTERLEAVED)
                    fbuf[b, r, pl.ds(k * 32, 16)] = lo
                    fbuf[b, r, pl.ds(k * 32 + 16, 16)] = hi
                return carry

            lax.fori_loop(0, CHUNK, cbody, 0)

        for b in range(LA):
            gather(b, b)
        for b in range(NB):
            j = b
            wait_gather(j, b)
            convert(b)
            scatter(j, b)
            if j + LA < NV:
                gather(j + LA, (j + LA) % NB)

        def body(i, carry):
            j0 = i * NB
            for b in range(NB):
                j = j0 + b
                wait_gather(j, b)
                wait_scatter(j - NB, b)
                convert(b)
                scatter(j, b)

                @pl.when(j + LA < NV)
                def _():
                    gather(j + LA, (j + LA) % NB)
            return carry

        lax.fori_loop(1, NV // NB, body, 0)
        for b in range(NB):
            wait_scatter(NV - NB + b, b)
        plsc.subcore_barrier()

        for r in range(WCHUNKS):
            b = r % 2
            sl = pl.ds(base + r * CHUNK, CHUNK)
            pltpu.sync_copy(acc_sh.at[sl], fbuf.at[b])
            pltpu.sync_copy(fbuf.at[b], out_hbm.at[cid, sl])

    return msg


def _pre_tc(N, D, H, NPAD, BN):
    def body(x_ref, w_ref, wq_ref, degp_ref, g_ref, gbf_ref, dinv_ref):
        deg = degp_ref[:, 0:1] + degp_ref[:, 1:2] + 1.0
        dinv = lax.rsqrt(deg)
        h = jnp.dot(x_ref[...], w_ref[...], preferred_element_type=F32)
        g_ref[...] = h * dinv
        hq = jnp.dot(x_ref[...].astype(jnp.bfloat16), wq_ref[...],
                     preferred_element_type=F32)
        gbf_ref[...] = (hq * dinv).astype(jnp.bfloat16)
        dinv_ref[...] = dinv

    return pl.pallas_call(
        body,
        grid=(N // BN,),
        in_specs=[
            pl.BlockSpec((BN, D), lambda i: (i, 0)),
            pl.BlockSpec((D, H), lambda i: (0, 0)),
            pl.BlockSpec((D, H), lambda i: (0, 0)),
            pl.BlockSpec((BN, NC), lambda i: (i, 0)),
        ],
        out_specs=[
            pl.BlockSpec((BN, H), lambda i: (i, 0)),
            pl.BlockSpec((BN, H), lambda i: (i, 0)),
            pl.BlockSpec((BN, 1), lambda i: (i, 0)),
        ],
        out_shape=[
            jax.ShapeDtypeStruct((N, H), F32),
            jax.ShapeDtypeStruct((N, H), jnp.bfloat16),
            jax.ShapeDtypeStruct((N, 1), F32),
        ],
    )


def _layer_tc(N, D, H, NPAD, BN):
    DH = D // NC

    def body(accp_ref, g_ref, dinv_ref, b_ref, w_ref, wq_ref,
             out_ref, gn_ref, gnbf_ref):
        dinv = dinv_ref[...]
        acc = jnp.concatenate([accp_ref[0], accp_ref[1]], axis=1)
        s = (acc + g_ref[...]) * dinv + b_ref[...]
        r = jnp.maximum(s, 0.0)
        nrm = jnp.sqrt(jnp.sum(r * r, axis=1, keepdims=True))
        o = r / jnp.maximum(nrm, 1e-12)
        out_ref[...] = o
        gn_ref[...] = jnp.dot(o, w_ref[...], preferred_element_type=F32) * dinv
        hq = jnp.dot(o.astype(jnp.bfloat16), wq_ref[...],
                     preferred_element_type=F32)
        gnbf_ref[...] = (hq * dinv).astype(jnp.bfloat16)

    return pl.pallas_call(
        body,
        grid=(N // BN,),
        in_specs=[
            pl.BlockSpec((NC, BN, DH), lambda i: (0, i, 0)),
            pl.BlockSpec((BN, H), lambda i: (i, 0)),
            pl.BlockSpec((BN, 1), lambda i: (i, 0)),
            pl.BlockSpec((1, H), lambda i: (0, 0)),
            pl.BlockSpec((H, H), lambda i: (0, 0)),
            pl.BlockSpec((H, H), lambda i: (0, 0)),
        ],
        out_specs=[
            pl.BlockSpec((BN, H), lambda i: (i, 0)),
            pl.BlockSpec((BN, H), lambda i: (i, 0)),
            pl.BlockSpec((BN, H), lambda i: (i, 0)),
        ],
        out_shape=[
            jax.ShapeDtypeStruct((N, H), F32),
            jax.ShapeDtypeStruct((N, H), F32),
            jax.ShapeDtypeStruct((N, H), jnp.bfloat16),
        ],
    )


def _final_tc(N, D, H, C, NPAD, BN):
    DH = D // NC

    def body(accp_ref, g_ref, dinv_ref, b_ref, o1_ref, o2_ref, wl_ref, bl_ref,
             out_ref):
        acc = jnp.concatenate([accp_ref[0], accp_ref[1]], axis=1)
        s = (acc + g_ref[...]) * dinv_ref[...] + b_ref[...]
        r = jnp.maximum(s, 0.0)
        nrm = jnp.sqrt(jnp.sum(r * r, axis=1, keepdims=True))
        o3 = r / jnp.maximum(nrm, 1e-12)
        logits = (
            jnp.dot(o1_ref[...], wl_ref[0:H], preferred_element_type=F32)
            + jnp.dot(o2_ref[...], wl_ref[H:2 * H], preferred_element_type=F32)
            + jnp.dot(o3, wl_ref[2 * H:3 * H], preferred_element_type=F32)
            + bl_ref[...]
        )
        m = jnp.max(logits, axis=1, keepdims=True)
        lse = jnp.log(jnp.sum(jnp.exp(logits - m), axis=1, keepdims=True)) + m
        out_ref[...] = logits - lse

    return pl.pallas_call(
        body,
        grid=(N // BN,),
        in_specs=[
            pl.BlockSpec((NC, BN, DH), lambda i: (0, i, 0)),
            pl.BlockSpec((BN, H), lambda i: (i, 0)),
            pl.BlockSpec((BN, 1), lambda i: (i, 0)),
            pl.BlockSpec((1, H), lambda i: (0, 0)),
            pl.BlockSpec((BN, H), lambda i: (i, 0)),
            pl.BlockSpec((BN, H), lambda i: (i, 0)),
            pl.BlockSpec((3 * H, C), lambda i: (0, 0)),
            pl.BlockSpec((1, C), lambda i: (0, 0)),
        ],
        out_specs=pl.BlockSpec((BN, C), lambda i: (i, 0)),
        out_shape=jax.ShapeDtypeStruct((N, C), F32),
    )


def kernel(x, edge_index, W1, b1, W2, b2, W3, b3, Wl, bl):
    N, D = x.shape
    H = W1.shape[1]
    C = Wl.shape[1]
    E = edge_index.shape[1]

    DH = D // NC
    TPB = -(-E // (NW * CHUNK))
    TPB = -(-TPB // 4) * 4
    TPB2 = 2 * TPB
    EPAD = NW * TPB * CHUNK
    NPAD = -(-(N + 1) // (NS * CHUNK)) * (NS * CHUNK)
    BN = 2000 if N % 2000 == 0 else 8

    src = edge_index[0]
    dst = edge_index[1]
    pad = EPAD - E
    if pad:
        ar = jnp.arange(pad, dtype=jnp.int32)
        src = jnp.concatenate([src, ar % N])
        dst = jnp.concatenate([dst, N + ar % (NPAD - N)])
    src_rs = src.reshape(NS, TPB2, CHUNK)
    dst_rs = dst.reshape(NS, TPB2, CHUNK)
    zeros_chunk = jnp.zeros((CHUNK, DH), F32)

    degp = _deg_kernel(NPAD, TPB)(dst.reshape(NW, TPB, CHUNK))
    degp_t = degp.T[:N]

    qidx = []
    for off in range(0, H, 32):
        for i in range(16):
            qidx += [off + i, off + 16 + i]
    qidx = jnp.asarray(qidx, dtype=jnp.int32)
    W1q = W1[:, qidx].astype(jnp.bfloat16)
    W2q = W2[:, qidx].astype(jnp.bfloat16)
    W3q = W3[:, qidx].astype(jnp.bfloat16)

    g1, g1bf, dinv = _pre_tc(N, D, H, NPAD, BN)(x, W1, W1q, degp_t)
    msg = _msg_kernel(N, D, NPAD, TPB2)
    layer = _layer_tc(N, D, H, NPAD, BN)

    def sc_view(g):
        return g.reshape(NC * N, DH)

    acc1 = msg(sc_view(g1bf), src_rs, dst_rs, zeros_chunk)
    out1, g2, g2bf = layer(acc1, g1, dinv, b1.reshape(1, H), W2, W2q)
    acc2 = msg(sc_view(g2bf), src_rs, dst_rs, zeros_chunk)
    out2, g3, g3bf = layer(acc2, g2, dinv, b2.reshape(1, H), W3, W3q)
    acc3 = msg(sc_view(g3bf), src_rs, dst_rs, zeros_chunk)
    return _final_tc(N, D, H, C, NPAD, BN)(
        acc3, g3, dinv, b3.reshape(1, H), out1, out2, Wl, bl.reshape(1, C)
    )


# speedup vs baseline: 1.8390x; 1.8390x over previous
import functools

import jax
import jax.numpy as jnp
from jax import lax
from jax.experimental import pallas as pl
from jax.experimental.pallas import tpu as pltpu
from jax.experimental.pallas import tpu_sc as plsc

NC = 2
NS = 16
NW = NC * NS
CHUNK = 128
F32 = jnp.float32


def _sc_mesh():
    return plsc.VectorSubcoreMesh(
        core_axis_name="c", subcore_axis_name="s", num_cores=NC, num_subcores=NS
    )


def _deg_kernel(NPAD, TPB):
    SEG = NPAD // NS

    @functools.partial(
        pl.kernel,
        out_type=jax.ShapeDtypeStruct((NC, NPAD), F32),
        mesh=_sc_mesh(),
        scratch_types=[
            pltpu.VMEM((TPB, CHUNK), jnp.int32),
            pltpu.VMEM((CHUNK,), F32),
            pltpu.VMEM((SEG,), F32),
            pltpu.VMEM_SHARED((NPAD,), F32),
            pltpu.SemaphoreType.DMA((8,)),
        ],
    )
    def deg(dst_hbm, out_hbm, dst_v, ones_v, seg_v, deg_sh, dsem):
        cid = lax.axis_index("c")
        sid = lax.axis_index("s")
        wid = cid * NS + sid
        pltpu.sync_copy(dst_hbm.at[wid], dst_v)

        zero16 = jnp.zeros((16,), F32)
        one16 = jnp.ones((16,), F32)
        for k in range(CHUNK // 16):
            ones_v[pl.ds(k * 16, 16)] = one16

        def zbody(q, carry):
            seg_v[pl.ds(q * 16, 16)] = zero16
            return carry

        lax.fori_loop(0, SEG // 16, zbody, 0)
        seg0 = sid * SEG
        sl = pl.ds(seg0, SEG)
        pltpu.sync_copy(seg_v, deg_sh.at[sl])
        plsc.subcore_barrier()

        WD = 8

        def dscat(j, b):
            pltpu.async_copy(ones_v, deg_sh.at[dst_v.at[j]], dsem.at[b],
                             add=True)

        def dwait(j, b):
            pltpu.make_async_copy(
                ones_v, deg_sh.at[dst_v.at[j]], dsem.at[b]
            ).wait()

        TPBW = TPB - TPB % WD

        for b in range(min(WD, TPBW)):
            dscat(b, b)

        def cbody(i, carry):
            j0 = i * WD
            for b in range(WD):
                j = j0 + b
                dwait(j, b)

                @pl.when(j + WD < TPBW)
                def _():
                    dscat(j + WD, b)
            return carry

        lax.fori_loop(0, TPBW // WD, cbody, 0)
        for j in range(TPBW, TPB):
            pltpu.sync_copy(ones_v, deg_sh.at[dst_v.at[j]], add=True)
        plsc.subcore_barrier()

        pltpu.sync_copy(deg_sh.at[sl], seg_v)
        pltpu.sync_copy(seg_v, out_hbm.at[cid, sl])

    return deg


def _msg_kernel(N, D, NPAD, TPB2):
    DH = D // NC
    RPT = NPAD // NS
    WCHUNKS = RPT // CHUNK

    PROBE_WIDE = False
    CH = 64 if PROBE_WIDE else CHUNK
    TW = D if PROBE_WIDE else D // NC
    NV = TPB2 * CHUNK // CH
    NB = 5
    LA = 3

    @functools.partial(
        pl.kernel,
        out_type=jax.ShapeDtypeStruct((NC, NPAD, DH), F32),
        mesh=_sc_mesh(),
        compiler_params=pltpu.CompilerParams(use_tc_tiling_on_sc=False),
        scratch_types=[
            pltpu.VMEM((NV, CH), jnp.int32),
            pltpu.VMEM((NV, CH), jnp.int32),
            pltpu.VMEM((NB, CH, TW), F32),
            pltpu.VMEM_SHARED((NPAD, DH), F32),
            pltpu.SemaphoreType.DMA((NB,)),
            pltpu.SemaphoreType.DMA((NB,)),
        ],
    )
    def msg(g_hbm, src_hbm, dst_hbm, zeros_hbm, out_hbm,
            src_v, dst_v, buf, acc_sh, gsem, ssem):
        cid = lax.axis_index("c")
        sid = lax.axis_index("s")
        gt = g_hbm

        pltpu.sync_copy(src_hbm.at[sid], src_v)
        pltpu.sync_copy(dst_hbm.at[sid], dst_v)

        cvec = jnp.full((16,), NC, jnp.int32)
        ovec = jnp.zeros((16,), jnp.int32) + cid

        def xbody(j, carry):
            for k in range(CH // 16):
                sl16 = pl.ds(k * 16, 16)
                src_v[j, sl16] = src_v[j, sl16] * cvec + ovec
            return carry

        if not PROBE_WIDE:
            lax.fori_loop(0, NV, xbody, 0)

        base = sid * RPT
        if not PROBE_WIDE:
            pltpu.sync_copy(zeros_hbm, buf.at[0])
            for r in range(WCHUNKS):
                pltpu.sync_copy(buf.at[0], acc_sh.at[pl.ds(base + r * CHUNK, CHUNK)])
            plsc.subcore_barrier()

        def gather(j, b):
            pltpu.async_copy(gt.at[src_v.at[j]], buf.at[b], gsem.at[b])

        def wait_gather(j, b):
            pltpu.make_async_copy(
                gt.at[src_v.at[j]], buf.at[b], gsem.at[b]
            ).wait()

        def scatter(j, b):
            if PROBE_WIDE:
                return
            pltpu.async_copy(
                buf.at[b], acc_sh.at[dst_v.at[j]], ssem.at[b], add=True
            )

        def wait_scatter(j, b):
            if PROBE_WIDE:
                return
            pltpu.make_async_copy(
                buf.at[b], acc_sh.at[dst_v.at[j]], ssem.at[b]
            ).wait()

        for b in range(LA):
            gather(b, b)
        for b in range(NB):
            j = b
            wait_gather(j, b)
            scatter(j, b)
            b2 = (b + LA) % NB
            if j + LA - NB >= 0:
                wait_scatter(j + LA - NB, b2)
            gather(j + LA, b2)

        def body(i, carry):
            j0 = i * NB
            for b in range(NB):
                j = j0 + b
                wait_gather(j, b)
                scatter(j, b)
                b2 = (b + LA) % NB

                @pl.when(j + LA < NV)
                def _():
                    wait_scatter(j + LA - NB, b2)
                    gather(j + LA, b2)
            return carry

        lax.fori_loop(1, NV // NB, body, 0)
        for b in range(NB):
            wait_scatter(NV - NB + b, b)
        plsc.subcore_barrier()

        if not PROBE_WIDE:
            for r in range(WCHUNKS):
                b = r % 2
                sl = pl.ds(base + r * CHUNK, CHUNK)
                pltpu.sync_copy(acc_sh.at[sl], buf.at[b])
                pltpu.sync_copy(buf.at[b], out_hbm.at[cid, sl])

    return msg


def _pre_tc(N, D, H, NPAD, BN):
    def body(x_ref, w_ref, degp_ref, g_ref, dinv_ref):
        deg = degp_ref[:, 0:1] + degp_ref[:, 1:2] + 1.0
        dinv = lax.rsqrt(deg)
        h = jnp.dot(x_ref[...], w_ref[...], preferred_element_type=F32)
        g_ref[...] = h * dinv
        dinv_ref[...] = dinv

    return pl.pallas_call(
        body,
        grid=(N // BN,),
        in_specs=[
            pl.BlockSpec((BN, D), lambda i: (i, 0)),
            pl.BlockSpec((D, H), lambda i: (0, 0)),
            pl.BlockSpec((BN, NC), lambda i: (i, 0)),
        ],
        out_specs=[
            pl.BlockSpec((BN, H), lambda i: (i, 0)),
            pl.BlockSpec((BN, 1), lambda i: (i, 0)),
        ],
        out_shape=[
            jax.ShapeDtypeStruct((N, H), F32),
            jax.ShapeDtypeStruct((N, 1), F32),
        ],
    )


def _layer_tc(N, D, H, NPAD, BN):
    DH = D // NC

    def body(accp_ref, g_ref, dinv_ref, b_ref, w_ref, out_ref, gn_ref):
        dinv = dinv_ref[...]
        acc = jnp.concatenate([accp_ref[0], accp_ref[1]], axis=1)
        s = (acc + g_ref[...]) * dinv + b_ref[...]
        r = jnp.maximum(s, 0.0)
        nrm = jnp.sqrt(jnp.sum(r * r, axis=1, keepdims=True))
        o = r / jnp.maximum(nrm, 1e-12)
        out_ref[...] = o
        gn_ref[...] = jnp.dot(o, w_ref[...], preferred_element_type=F32) * dinv

    return pl.pallas_call(
        body,
        grid=(N // BN,),
        in_specs=[
            pl.BlockSpec((NC, BN, DH), lambda i: (0, i, 0)),
            pl.BlockSpec((BN, H), lambda i: (i, 0)),
            pl.BlockSpec((BN, 1), lambda i: (i, 0)),
            pl.BlockSpec((1, H), lambda i: (0, 0)),
            pl.BlockSpec((H, H), lambda i: (0, 0)),
        ],
        out_specs=[
            pl.BlockSpec((BN, H), lambda i: (i, 0)),
            pl.BlockSpec((BN, H), lambda i: (i, 0)),
        ],
        out_shape=[
            jax.ShapeDtypeStruct((N, H), F32),
            jax.ShapeDtypeStruct((N, H), F32),
        ],
    )


def _final_tc(N, D, H, C, NPAD, BN):
    DH = D // NC

    def body(accp_ref, g_ref, dinv_ref, b_ref, o1_ref, o2_ref, wl_ref, bl_ref,
             out_ref):
        acc = jnp.concatenate([accp_ref[0], accp_ref[1]], axis=1)
        s = (acc + g_ref[...]) * dinv_ref[...] + b_ref[...]
        r = jnp.maximum(s, 0.0)
        nrm = jnp.sqrt(jnp.sum(r * r, axis=1, keepdims=True))
        o3 = r / jnp.maximum(nrm, 1e-12)
        logits = (
            jnp.dot(o1_ref[...], wl_ref[0:H], preferred_element_type=F32)
            + jnp.dot(o2_ref[...], wl_ref[H:2 * H], preferred_element_type=F32)
            + jnp.dot(o3, wl_ref[2 * H:3 * H], preferred_element_type=F32)
            + bl_ref[...]
        )
        m = jnp.max(logits, axis=1, keepdims=True)
        lse = jnp.log(jnp.sum(jnp.exp(logits - m), axis=1, keepdims=True)) + m
        out_ref[...] = logits - lse

    return pl.pallas_call(
        body,
        grid=(N // BN,),
        in_specs=[
            pl.BlockSpec((NC, BN, DH), lambda i: (0, i, 0)),
            pl.BlockSpec((BN, H), lambda i: (i, 0)),
            pl.BlockSpec((BN, 1), lambda i: (i, 0)),
            pl.BlockSpec((1, H), lambda i: (0, 0)),
            pl.BlockSpec((BN, H), lambda i: (i, 0)),
            pl.BlockSpec((BN, H), lambda i: (i, 0)),
            pl.BlockSpec((3 * H, C), lambda i: (0, 0)),
            pl.BlockSpec((1, C), lambda i: (0, 0)),
        ],
        out_specs=pl.BlockSpec((BN, C), lambda i: (i, 0)),
        out_shape=jax.ShapeDtypeStruct((N, C), F32),
    )


def kernel(x, edge_index, W1, b1, W2, b2, W3, b3, Wl, bl):
    N, D = x.shape
    H = W1.shape[1]
    C = Wl.shape[1]
    E = edge_index.shape[1]

    DH = D // NC
    TPB = -(-E // (NW * CHUNK))
    TPB = -(-TPB // 4) * 4
    TPB2 = 2 * TPB
    EPAD = NW * TPB * CHUNK
    NPAD = -(-(N + 1) // (NS * CHUNK)) * (NS * CHUNK)
    BN = 2000 if N % 2000 == 0 else 8

    src = edge_index[0]
    dst = edge_index[1]
    pad = EPAD - E
    if pad:
        ar = jnp.arange(pad, dtype=jnp.int32)
        src = jnp.concatenate([src, ar % N])
        dst = jnp.concatenate([dst, N + ar % (NPAD - N)])
    src_rs = src.reshape(NS, TPB2, CHUNK)
    dst_rs = dst.reshape(NS, TPB2, CHUNK)
    zeros_chunk = jnp.zeros((CHUNK, DH), F32)

    degp = _deg_kernel(NPAD, TPB)(dst.reshape(NW, TPB, CHUNK))
    degp_t = degp.T[:N]

    g1, dinv = _pre_tc(N, D, H, NPAD, BN)(x, W1, degp_t)
    msg = _msg_kernel(N, D, NPAD, TPB2)
    layer = _layer_tc(N, D, H, NPAD, BN)

    def sc_view(g):
        return g.reshape(NC * N, DH)

    acc1 = msg(sc_view(g1), src_rs, dst_rs, zeros_chunk)
    out1, g2 = layer(acc1, g1, dinv, b1.reshape(1, H), W2)
    acc2 = msg(sc_view(g2), src_rs, dst_rs, zeros_chunk)
    out2, g3 = layer(acc2, g2, dinv, b2.reshape(1, H), W3)
    acc3 = msg(sc_view(g3), src_rs, dst_rs, zeros_chunk)
    return _final_tc(N, D, H, C, NPAD, BN)(
        acc3, g3, dinv, b3.reshape(1, H), out1, out2, Wl, bl.reshape(1, C)
    )
